# baseline (speedup 1.0000x reference)
import jax
import jax.numpy as jnp
from jax import lax
from jax.experimental import pallas as pl
from jax.experimental.pallas import tpu as pltpu
from jax.experimental.pallas import tpu_sc as plsc

B, S, D = 1, 2048, 768
DFF = 3072
E = 8
DH = 768
TEMP = 1.0
EPS = 1e-5

T_BLK = 256
NB = S // T_BLK
G_BLK = 256
NSLOT = 2 * S + E * G_BLK
ND = NSLOT // G_BLK
NW = 32
TPW = S // NW
BF = jnp.bfloat16
F32 = jnp.float32
I32 = jnp.int32


def _route_body(x_ref, lng_ref, lnb_ref, wr_ref, br_ref,
                h_ref, w1_ref, w2_ref, e0_ref, e1_ref, r0_ref, r1_ref,
                cnt_ref, run_ref):
    @pl.when(pl.program_id(0) == 0)
    def _():
        run_ref[...] = jnp.zeros((1, 16), F32)

    xb = x_ref[...]
    mu = jnp.mean(xb, axis=-1, keepdims=True)
    var = jnp.mean((xb - mu) ** 2, axis=-1, keepdims=True)
    hb = (xb - mu) / jnp.sqrt(var + EPS) * lng_ref[...] + lnb_ref[...]
    h_ref[...] = hb

    logits = jnp.dot(hb.astype(BF), wr_ref[...],
                     preferred_element_type=F32) + br_ref[...]
    idx = lax.broadcasted_iota(I32, (T_BLK, E), 1)
    m1 = jnp.max(logits, axis=-1, keepdims=True)
    i1 = jnp.min(jnp.where(logits == m1, idx, E), axis=-1, keepdims=True)
    mask1 = idx == i1
    rest = jnp.where(mask1, -jnp.inf, logits)
    m2 = jnp.max(rest, axis=-1, keepdims=True)
    i2 = jnp.min(jnp.where(rest == m2, idx, E), axis=-1, keepdims=True)
    mask2 = idx == i2
    w1 = 1.0 / (1.0 + jnp.exp((m2 - m1) / TEMP))

    m1f = mask1.astype(F32)
    m2f = mask2.astype(F32)
    mm = m1f + m2f
    ri = lax.broadcasted_iota(I32, (T_BLK, T_BLK), 0)
    ci = lax.broadcasted_iota(I32, (T_BLK, T_BLK), 1)
    ltri = (ci < ri).astype(BF)
    csum = jnp.dot(ltri, mm.astype(BF), preferred_element_type=F32)
    run = run_ref[...][:, :E]
    cg = csum + run
    r0 = jnp.sum(jnp.where(mask1, cg, 0.0), axis=-1)
    r1 = jnp.sum(jnp.where(mask2, cg + m1f, 0.0), axis=-1)
    newrun = run + jnp.sum(mm, axis=0, keepdims=True)
    run_ref[:, :E] = newrun
    cnt_ref[:, :E] = newrun.astype(I32)
    cnt_ref[:, E:] = jnp.zeros((1, 16 - E), I32)

    w1_ref[...] = w1.reshape(1, 1, T_BLK)
    w2_ref[...] = (1.0 - w1).reshape(1, 1, T_BLK)
    e0_ref[...] = i1.astype(I32).reshape(1, 1, T_BLK)
    e1_ref[...] = i2.astype(I32).reshape(1, 1, T_BLK)
    r0_ref[...] = r0.astype(I32).reshape(1, 1, T_BLK)
    r1_ref[...] = r1.astype(I32).reshape(1, 1, T_BLK)


def _stage_a(x, ln_g, ln_b, wr, br):
    full = lambda shape: pl.BlockSpec(shape, lambda t: (0,) * len(shape))
    per_tok = lambda dt: jax.ShapeDtypeStruct((NB, 1, T_BLK), dt)
    return pl.pallas_call(
        _route_body,
        grid=(NB,),
        in_specs=[
            pl.BlockSpec((T_BLK, D), lambda t: (t, 0)),
            full((1, D)), full((1, D)), full((D, E)), full((1, E)),
        ],
        out_specs=[
            pl.BlockSpec((T_BLK, D), lambda t: (t, 0)),
            pl.BlockSpec((1, 1, T_BLK), lambda t: (t, 0, 0)),
            pl.BlockSpec((1, 1, T_BLK), lambda t: (t, 0, 0)),
            pl.BlockSpec((1, 1, T_BLK), lambda t: (t, 0, 0)),
            pl.BlockSpec((1, 1, T_BLK), lambda t: (t, 0, 0)),
            pl.BlockSpec((1, 1, T_BLK), lambda t: (t, 0, 0)),
            pl.BlockSpec((1, 1, T_BLK), lambda t: (t, 0, 0)),
            full((1, 16)),
        ],
        out_shape=[
            jax.ShapeDtypeStruct((S, D), F32),
            per_tok(F32), per_tok(F32),
            per_tok(I32), per_tok(I32), per_tok(I32), per_tok(I32),
            jax.ShapeDtypeStruct((1, 16), I32),
        ],
        scratch_shapes=[pltpu.VMEM((1, 16), F32)],
        compiler_params=pltpu.CompilerParams(
            dimension_semantics=("arbitrary",)),
    )(x, ln_g.reshape(1, D), ln_b.reshape(1, D), wr, br.reshape(1, E))


def _shared_body(x_ref, h_ref, w1_ref, b1_ref, w2_ref, b2_ref, o_ref):
    hbb = h_ref[...].astype(BF)
    t1 = jax.nn.gelu(jnp.dot(hbb, w1_ref[...],
                             preferred_element_type=F32) + b1_ref[...])
    sh = jnp.dot(t1.astype(BF), w2_ref[...],
                 preferred_element_type=F32) + b2_ref[...]
    o_ref[...] = x_ref[...] + sh


def _stage_b(x, h, wfc1, bfc1, wfc2, bfc2):
    full = lambda shape: pl.BlockSpec(shape, lambda t: (0,) * len(shape))
    return pl.pallas_call(
        _shared_body,
        grid=(NB,),
        in_specs=[
            pl.BlockSpec((T_BLK, D), lambda t: (t, 0)),
            pl.BlockSpec((T_BLK, D), lambda t: (t, 0)),
            full((D, DFF)), full((1, DFF)), full((DFF, D)), full((1, D)),
        ],
        out_specs=pl.BlockSpec((T_BLK, D), lambda t: (t, 0)),
        out_shape=jax.ShapeDtypeStruct((S, D), F32),
        compiler_params=pltpu.CompilerParams(
            dimension_semantics=("arbitrary",)),
    )(x, h, wfc1, bfc1.reshape(1, DFF), wfc2, bfc2.reshape(1, D))


def _slots_body(cnt_ref, e0_ref, e1_ref, r0_ref, r1_ref,
                s0_ref, s1_ref, blk_ref, na_ref):
    cntb = cnt_ref[...][:, :E].astype(F32)
    pb = jnp.floor((cntb + (G_BLK - 1)) * (1.0 / G_BLK))
    ri = lax.broadcasted_iota(I32, (E, E), 0)
    ci = lax.broadcasted_iota(I32, (E, E), 1)
    utri = (ri < ci).astype(BF)
    baseb = jnp.dot(pb.astype(BF), utri, preferred_element_type=F32)

    e0 = e0_ref[...].reshape(1, T_BLK)
    e1 = e1_ref[...].reshape(1, T_BLK)
    s0 = r0_ref[...].reshape(1, T_BLK).astype(F32)
    s1 = r1_ref[...].reshape(1, T_BLK).astype(F32)
    iota8 = lax.broadcasted_iota(I32, (1, E), 1)
    bstart = lax.broadcasted_iota(I32, (1, NW), 1).astype(F32)
    blk = jnp.zeros((1, NW), F32)
    na = jnp.zeros((1, 16), F32)
    for e in range(E):
        be = jnp.sum(jnp.where(iota8 == e, baseb, 0.0), axis=-1,
                     keepdims=True)
        pe = jnp.sum(jnp.where(iota8 == e, pb, 0.0), axis=-1, keepdims=True)
        s0 = s0 + jnp.where(e0 == e, be * G_BLK, 0.0)
        s1 = s1 + jnp.where(e1 == e, be * G_BLK, 0.0)
        blk = blk + jnp.where(bstart >= be, 1.0, 0.0)
        na = na + pe
    s0_ref[...] = s0.astype(I32).reshape(1, 1, T_BLK)
    s1_ref[...] = s1.astype(I32).reshape(1, 1, T_BLK)
    blk_ref[...] = (blk - 1.0).astype(I32)
    na_ref[...] = na.astype(I32)


def _stage_a2(cnts, e0o, e1o, r0o, r1o):
    full = lambda shape: pl.BlockSpec(shape, lambda t: (0,) * len(shape))
    tok = pl.BlockSpec((1, 1, T_BLK), lambda t: (t, 0, 0))
    return pl.pallas_call(
        _slots_body,
        grid=(NB,),
        in_specs=[full((1, 16)), tok, tok, tok, tok],
        out_specs=[tok, tok, full((1, NW)), full((1, 16))],
        out_shape=[
            jax.ShapeDtypeStruct((NB, 1, T_BLK), I32),
            jax.ShapeDtypeStruct((NB, 1, T_BLK), I32),
            jax.ShapeDtypeStruct((1, NW), I32),
            jax.ShapeDtypeStruct((1, 16), I32),
        ],
        compiler_params=pltpu.CompilerParams(
            dimension_semantics=("arbitrary",)),
    )(cnts, e0o, e1o, r0o, r1o)


def _dispatch_body(h_hbm, s0_hbm, s1_hbm, hperm_hbm,
                   s0v, s1v, hrows, sem0, sem1):
    wid = lax.axis_index("s") * 2 + lax.axis_index("c")
    base = wid * TPW
    pltpu.sync_copy(s0_hbm.at[pl.ds(base, TPW)], s0v)
    pltpu.sync_copy(s1_hbm.at[pl.ds(base, TPW)], s1v)
    pltpu.sync_copy(h_hbm.at[pl.ds(base, TPW)], hrows)
    d0 = pltpu.async_copy(hrows, hperm_hbm.at[s0v], sem0)
    d1 = pltpu.async_copy(hrows, hperm_hbm.at[s1v], sem1)
    d0.wait()
    d1.wait()


def _stage_c(h, s0, s1):
    mesh = plsc.VectorSubcoreMesh(core_axis_name="c", subcore_axis_name="s", num_cores=2, num_subcores=16)
    f = pl.kernel(
        _dispatch_body,
        out_type=jax.ShapeDtypeStruct((NSLOT, D), F32),
        mesh=mesh,
        compiler_params=pltpu.CompilerParams(needs_layout_passes=False),
        scratch_types=[
            pltpu.VMEM((TPW,), I32), pltpu.VMEM((TPW,), I32),
            pltpu.VMEM((TPW, D), F32),
            pltpu.SemaphoreType.DMA, pltpu.SemaphoreType.DMA,
        ],
    )
    return f(h, s0, s1)


def _group_body(blk_sref, na_sref, hp_ref, w1_ref, b1_ref, w2_ref, b2_ref,
                y_ref):
    b = pl.program_id(0)

    @pl.when(b < na_sref[0])
    def _():
        xb = hp_ref[...].astype(BF)
        a1 = jax.nn.gelu(jnp.dot(xb, w1_ref[0],
                                 preferred_element_type=F32) + b1_ref[0, 0])
        y = jnp.dot(a1.astype(BF), w2_ref[0],
                    preferred_element_type=F32) + b2_ref[0, 0]
        y_ref[...] = y


def _stage_d(blk, na, hperm, we1, be1, we2, be2):
    def beff(b, blk_r, na_r):
        return jnp.minimum(b, na_r[0] - 1)

    grid_spec = pltpu.PrefetchScalarGridSpec(
        num_scalar_prefetch=2,
        grid=(ND,),
        in_specs=[
            pl.BlockSpec((G_BLK, D), lambda b, bl, na: (beff(b, bl, na), 0)),
            pl.BlockSpec((1, D, DH),
                         lambda b, bl, na: (bl[beff(b, bl, na)], 0, 0)),
            pl.BlockSpec((1, 1, DH),
                         lambda b, bl, na: (bl[beff(b, bl, na)], 0, 0)),
            pl.BlockSpec((1, DH, D),
                         lambda b, bl, na: (bl[beff(b, bl, na)], 0, 0)),
            pl.BlockSpec((1, 1, D),
                         lambda b, bl, na: (bl[beff(b, bl, na)], 0, 0)),
        ],
        out_specs=pl.BlockSpec((G_BLK, D),
                               lambda b, bl, na: (beff(b, bl, na), 0)),
    )
    return pl.pallas_call(
        _group_body,
        grid_spec=grid_spec,
        out_shape=jax.ShapeDtypeStruct((NSLOT, D), F32),
        compiler_params=pltpu.CompilerParams(
            dimension_semantics=("arbitrary",)),
    )(blk, na, hperm, we1, be1.reshape(E, 1, DH), we2, be2.reshape(E, 1, D))


def _combine_body(o1_hbm, yp_hbm, s0_hbm, s1_hbm, w0_hbm, w1_hbm, out_hbm,
                  obuf, y0buf, y1buf, s0v, s1v, w0v, w1v, sem0, sem1):
    wid = lax.axis_index("s") * 2 + lax.axis_index("c")
    base = wid * TPW
    half = TPW // 2

    for hf in range(2):
        r0 = base + hf * half
        pltpu.sync_copy(o1_hbm.at[pl.ds(r0, half)], obuf)
        pltpu.sync_copy(s0_hbm.at[pl.ds(r0, half)], s0v)
        pltpu.sync_copy(s1_hbm.at[pl.ds(r0, half)], s1v)
        pltpu.sync_copy(w0_hbm.at[pl.ds(r0, half)], w0v)
        pltpu.sync_copy(w1_hbm.at[pl.ds(r0, half)], w1v)
        g0 = pltpu.async_copy(yp_hbm.at[s0v], y0buf, sem0)
        g1 = pltpu.async_copy(yp_hbm.at[s1v], y1buf, sem1)
        g0.wait()
        g1.wait()

        def row_body(i, _):
            a = plsc.load_gather(w0v, [jnp.full((16,), i, I32)])
            bb = plsc.load_gather(w1v, [jnp.full((16,), i, I32)])

            def col_body(c, _):
                sl = pl.ds(pl.multiple_of(c * 16, 16), 16)
                obuf[i, sl] = (obuf[i, sl] + a * y0buf[i, sl]
                               + bb * y1buf[i, sl])
                return 0

            lax.fori_loop(0, D // 16, col_body, 0)
            return 0

        lax.fori_loop(0, half, row_body, 0)
        pltpu.sync_copy(obuf, out_hbm.at[pl.ds(r0, half)])


def _stage_e(out1, yperm, s0, s1, w1, w2):
    mesh = plsc.VectorSubcoreMesh(core_axis_name="c", subcore_axis_name="s", num_cores=2, num_subcores=16)
    half = TPW // 2
    f = pl.kernel(
        _combine_body,
        out_type=jax.ShapeDtypeStruct((S, D), F32),
        mesh=mesh,
        compiler_params=pltpu.CompilerParams(needs_layout_passes=False),
        scratch_types=[
            pltpu.VMEM((half, D), F32), pltpu.VMEM((half, D), F32),
            pltpu.VMEM((half, D), F32),
            pltpu.VMEM((half,), I32), pltpu.VMEM((half,), I32),
            pltpu.VMEM((half,), F32), pltpu.VMEM((half,), F32),
            pltpu.SemaphoreType.DMA, pltpu.SemaphoreType.DMA,
        ],
    )
    return f(out1, yperm, s0, s1, w1, w2)


@jax.jit
def kernel(hidden_states, ln_g, ln_b, W_fc1, b_fc1, W_fc2, b_fc2,
           W_router, b_router, We1, be1, We2, be2):
    x = hidden_states.reshape(S, D)
    (h, w1o, w2o, e0o, e1o, r0o, r1o, cnts) = _stage_a(
        x, ln_g, ln_b, W_router.astype(BF), b_router)
    w1 = w1o.reshape(S)
    w2 = w2o.reshape(S)

    s0o, s1o, blk, na = _stage_a2(cnts, e0o, e1o, r0o, r1o)
    s0 = s0o.reshape(S)
    s1 = s1o.reshape(S)
    hperm = _stage_c(h, s0, s1)
    out1 = _stage_b(x, h, W_fc1.astype(BF), b_fc1, W_fc2.astype(BF), b_fc2)
    yperm = _stage_d(blk.reshape(NW), na.reshape(16), hperm,
                     We1.astype(BF), be1, We2.astype(BF), be2)
    out = _stage_e(out1, yperm, s0, s1, w1, w2)
    return out.reshape(B, S, D)


# speedup vs baseline: 1.2337x; 1.2337x over previous
import jax
import jax.numpy as jnp
from jax import lax
from jax.experimental import pallas as pl
from jax.experimental.pallas import tpu as pltpu
from jax.experimental.pallas import tpu_sc as plsc

B, S, D = 1, 2048, 768
DFF = 3072
E = 8
DH = 768
TEMP = 1.0
EPS = 1e-5

T_BLK = 256
NB = S // T_BLK
G_BLK = 256
NSLOT = 2 * S + E * G_BLK
ND = NSLOT // G_BLK
NW = 32
TPW = S // NW
BF = jnp.bfloat16
F32 = jnp.float32
I32 = jnp.int32


def _route_body(x_ref, lng_ref, lnb_ref, wr_ref, br_ref,
                h_ref, w1_ref, w2_ref, e0_ref, e1_ref, r0_ref, r1_ref,
                cnt_ref, run_ref):
    @pl.when(pl.program_id(0) == 0)
    def _():
        run_ref[...] = jnp.zeros((1, 16), F32)

    xb = x_ref[...]
    mu = jnp.mean(xb, axis=-1, keepdims=True)
    var = jnp.mean((xb - mu) ** 2, axis=-1, keepdims=True)
    hb = (xb - mu) / jnp.sqrt(var + EPS) * lng_ref[...] + lnb_ref[...]
    h_ref[...] = hb

    logits = jnp.dot(hb.astype(BF), wr_ref[...].astype(BF),
                     preferred_element_type=F32) + br_ref[...]
    idx = lax.broadcasted_iota(I32, (T_BLK, E), 1)
    m1 = jnp.max(logits, axis=-1, keepdims=True)
    i1 = jnp.min(jnp.where(logits == m1, idx, E), axis=-1, keepdims=True)
    mask1 = idx == i1
    rest = jnp.where(mask1, -jnp.inf, logits)
    m2 = jnp.max(rest, axis=-1, keepdims=True)
    i2 = jnp.min(jnp.where(rest == m2, idx, E), axis=-1, keepdims=True)
    mask2 = idx == i2
    w1 = 1.0 / (1.0 + jnp.exp((m2 - m1) / TEMP))

    m1f = mask1.astype(F32)
    m2f = mask2.astype(F32)
    mm = m1f + m2f
    ri = lax.broadcasted_iota(I32, (T_BLK, T_BLK), 0)
    ci = lax.broadcasted_iota(I32, (T_BLK, T_BLK), 1)
    ltri = (ci < ri).astype(BF)
    csum = jnp.dot(ltri, mm.astype(BF), preferred_element_type=F32)
    run = run_ref[...][:, :E]
    cg = csum + run
    r0 = jnp.sum(jnp.where(mask1, cg, 0.0), axis=-1)
    r1 = jnp.sum(jnp.where(mask2, cg + m1f, 0.0), axis=-1)
    newrun = run + jnp.sum(mm, axis=0, keepdims=True)
    run_ref[:, :E] = newrun
    cnt_ref[:, :E] = newrun.astype(I32)
    cnt_ref[:, E:] = jnp.zeros((1, 16 - E), I32)

    ones128 = jnp.ones((1, 128), F32)
    w1_ref[...] = w1 * ones128
    w2_ref[...] = (1.0 - w1) * ones128
    e0_ref[...] = i1.astype(I32).reshape(1, 1, T_BLK)
    e1_ref[...] = i2.astype(I32).reshape(1, 1, T_BLK)
    r0_ref[...] = r0.astype(I32).reshape(1, 1, T_BLK)
    r1_ref[...] = r1.astype(I32).reshape(1, 1, T_BLK)


def _stage_a(x, ln_g, ln_b, wr, br):
    full = lambda shape: pl.BlockSpec(shape, lambda t: (0,) * len(shape))
    per_tok = lambda dt: jax.ShapeDtypeStruct((NB, 1, T_BLK), dt)
    return pl.pallas_call(
        _route_body,
        grid=(NB,),
        in_specs=[
            pl.BlockSpec((T_BLK, D), lambda t: (t, 0)),
            full((1, D)), full((1, D)), full((D, E)), full((1, E)),
        ],
        out_specs=[
            pl.BlockSpec((T_BLK, D), lambda t: (t, 0)),
            pl.BlockSpec((T_BLK, 128), lambda t: (t, 0)),
            pl.BlockSpec((T_BLK, 128), lambda t: (t, 0)),
            pl.BlockSpec((1, 1, T_BLK), lambda t: (t, 0, 0)),
            pl.BlockSpec((1, 1, T_BLK), lambda t: (t, 0, 0)),
            pl.BlockSpec((1, 1, T_BLK), lambda t: (t, 0, 0)),
            pl.BlockSpec((1, 1, T_BLK), lambda t: (t, 0, 0)),
            full((1, 16)),
        ],
        out_shape=[
            jax.ShapeDtypeStruct((S, D), F32),
            jax.ShapeDtypeStruct((S, 128), F32),
            jax.ShapeDtypeStruct((S, 128), F32),
            per_tok(I32), per_tok(I32), per_tok(I32), per_tok(I32),
            jax.ShapeDtypeStruct((1, 16), I32),
        ],
        scratch_shapes=[pltpu.VMEM((1, 16), F32)],
        compiler_params=pltpu.CompilerParams(
            dimension_semantics=("arbitrary",)),
    )(x, ln_g.reshape(1, D), ln_b.reshape(1, D), wr, br.reshape(1, E))


def _shared_body(x_ref, h_ref, w1_ref, b1_ref, w2_ref, b2_ref, o_ref):
    hbb = h_ref[...].astype(BF)
    t1 = jnp.dot(hbb, w1_ref[...].astype(BF),
                 preferred_element_type=F32) + b1_ref[...]
    a1 = jax.nn.gelu(t1.astype(BF))
    sh = jnp.dot(a1, w2_ref[...].astype(BF),
                 preferred_element_type=F32) + b2_ref[...]
    o_ref[...] = x_ref[...] + sh


def _stage_b(x, h, wfc1, bfc1, wfc2, bfc2):
    full = lambda shape: pl.BlockSpec(shape, lambda t: (0,) * len(shape))
    return pl.pallas_call(
        _shared_body,
        grid=(NB,),
        in_specs=[
            pl.BlockSpec((T_BLK, D), lambda t: (t, 0)),
            pl.BlockSpec((T_BLK, D), lambda t: (t, 0)),
            full((D, DFF)), full((1, DFF)), full((DFF, D)), full((1, D)),
        ],
        out_specs=pl.BlockSpec((T_BLK, D), lambda t: (t, 0)),
        out_shape=jax.ShapeDtypeStruct((S, D), F32),
        compiler_params=pltpu.CompilerParams(
            dimension_semantics=("arbitrary",)),
    )(x, h, wfc1, bfc1.reshape(1, DFF), wfc2, bfc2.reshape(1, D))


def _slots_body(cnt_ref, e0_ref, e1_ref, r0_ref, r1_ref,
                s0_ref, s1_ref, blk_ref, na_ref):
    cntb = cnt_ref[...][:, :E].astype(F32)
    pb = jnp.floor((cntb + (G_BLK - 1)) * (1.0 / G_BLK))
    ri = lax.broadcasted_iota(I32, (E, E), 0)
    ci = lax.broadcasted_iota(I32, (E, E), 1)
    utri = (ri < ci).astype(BF)
    baseb = jnp.dot(pb.astype(BF), utri, preferred_element_type=F32)

    e0 = e0_ref[...].reshape(1, T_BLK)
    e1 = e1_ref[...].reshape(1, T_BLK)
    s0 = r0_ref[...].reshape(1, T_BLK).astype(F32)
    s1 = r1_ref[...].reshape(1, T_BLK).astype(F32)
    iota8 = lax.broadcasted_iota(I32, (1, E), 1)
    bstart = lax.broadcasted_iota(I32, (1, NW), 1).astype(F32)
    blk = jnp.zeros((1, NW), F32)
    na = jnp.zeros((1, 16), F32)
    for e in range(E):
        be = jnp.sum(jnp.where(iota8 == e, baseb, 0.0), axis=-1,
                     keepdims=True)
        pe = jnp.sum(jnp.where(iota8 == e, pb, 0.0), axis=-1, keepdims=True)
        s0 = s0 + jnp.where(e0 == e, be * G_BLK, 0.0)
        s1 = s1 + jnp.where(e1 == e, be * G_BLK, 0.0)
        blk = blk + jnp.where(bstart >= be, 1.0, 0.0)
        na = na + pe
    s0_ref[...] = s0.astype(I32).reshape(1, 1, T_BLK)
    s1_ref[...] = s1.astype(I32).reshape(1, 1, T_BLK)
    blk_ref[...] = (blk - 1.0).astype(I32)
    na_ref[...] = na.astype(I32)


def _stage_a2(cnts, e0o, e1o, r0o, r1o):
    full = lambda shape: pl.BlockSpec(shape, lambda t: (0,) * len(shape))
    tok = pl.BlockSpec((1, 1, T_BLK), lambda t: (t, 0, 0))
    return pl.pallas_call(
        _slots_body,
        grid=(NB,),
        in_specs=[full((1, 16)), tok, tok, tok, tok],
        out_specs=[tok, tok, full((1, NW)), full((1, 16))],
        out_shape=[
            jax.ShapeDtypeStruct((NB, 1, T_BLK), I32),
            jax.ShapeDtypeStruct((NB, 1, T_BLK), I32),
            jax.ShapeDtypeStruct((1, NW), I32),
            jax.ShapeDtypeStruct((1, 16), I32),
        ],
        compiler_params=pltpu.CompilerParams(
            dimension_semantics=("arbitrary",)),
    )(cnts, e0o, e1o, r0o, r1o)


def _dispatch_body(h_hbm, wr0_hbm, wr1_hbm, s0_hbm, s1_hbm,
                   hperm_hbm, wperm_hbm,
                   s0v, s1v, hrows, w0rows, w1rows,
                   sem0, sem1, sem2, sem3):
    wid = lax.axis_index("s") * 2 + lax.axis_index("c")
    base = wid * TPW
    pltpu.sync_copy(s0_hbm.at[pl.ds(base, TPW)], s0v)
    pltpu.sync_copy(s1_hbm.at[pl.ds(base, TPW)], s1v)
    pltpu.sync_copy(h_hbm.at[pl.ds(base, TPW)], hrows)
    pltpu.sync_copy(wr0_hbm.at[pl.ds(base, TPW)], w0rows)
    pltpu.sync_copy(wr1_hbm.at[pl.ds(base, TPW)], w1rows)
    d0 = pltpu.async_copy(hrows, hperm_hbm.at[s0v], sem0)
    d1 = pltpu.async_copy(hrows, hperm_hbm.at[s1v], sem1)
    d2 = pltpu.async_copy(w0rows, wperm_hbm.at[s0v], sem2)
    d3 = pltpu.async_copy(w1rows, wperm_hbm.at[s1v], sem3)
    d0.wait()
    d1.wait()
    d2.wait()
    d3.wait()


def _stage_c(h, wr0, wr1, s0, s1):
    mesh = plsc.VectorSubcoreMesh(core_axis_name="c", subcore_axis_name="s", num_cores=2, num_subcores=16)
    f = pl.kernel(
        _dispatch_body,
        out_type=[
            jax.ShapeDtypeStruct((NSLOT, D), F32),
            jax.ShapeDtypeStruct((NSLOT, 128), F32),
        ],
        mesh=mesh,
        compiler_params=pltpu.CompilerParams(needs_layout_passes=False),
        scratch_types=[
            pltpu.VMEM((TPW,), I32), pltpu.VMEM((TPW,), I32),
            pltpu.VMEM((TPW, D), F32),
            pltpu.VMEM((TPW, 128), F32), pltpu.VMEM((TPW, 128), F32),
            pltpu.SemaphoreType.DMA, pltpu.SemaphoreType.DMA,
            pltpu.SemaphoreType.DMA, pltpu.SemaphoreType.DMA,
        ],
    )
    return f(h, wr0, wr1, s0, s1)


def _group_body(blk_sref, na_sref, hp_ref, wp_ref, w1_ref, b1_ref, w2_ref,
                b2_ref, y_ref):
    b = pl.program_id(0)

    @pl.when(b < na_sref[0])
    def _():
        xb = hp_ref[...].astype(BF)
        t1 = jnp.dot(xb, w1_ref[0].astype(BF),
                     preferred_element_type=F32) + b1_ref[0, 0]
        a1 = jax.nn.gelu(t1.astype(BF))
        y = jnp.dot(a1, w2_ref[0].astype(BF),
                    preferred_element_type=F32) + b2_ref[0, 0]
        y_ref[...] = y * wp_ref[...][:, 0:1]


def _stage_d(blk, na, hperm, wperm, we1, be1, we2, be2):
    def beff(b, blk_r, na_r):
        return jnp.minimum(b, na_r[0] - 1)

    grid_spec = pltpu.PrefetchScalarGridSpec(
        num_scalar_prefetch=2,
        grid=(ND,),
        in_specs=[
            pl.BlockSpec((G_BLK, D), lambda b, bl, na: (beff(b, bl, na), 0)),
            pl.BlockSpec((G_BLK, 128),
                         lambda b, bl, na: (beff(b, bl, na), 0)),
            pl.BlockSpec((1, D, DH),
                         lambda b, bl, na: (bl[beff(b, bl, na)], 0, 0)),
            pl.BlockSpec((1, 1, DH),
                         lambda b, bl, na: (bl[beff(b, bl, na)], 0, 0)),
            pl.BlockSpec((1, DH, D),
                         lambda b, bl, na: (bl[beff(b, bl, na)], 0, 0)),
            pl.BlockSpec((1, 1, D),
                         lambda b, bl, na: (bl[beff(b, bl, na)], 0, 0)),
        ],
        out_specs=pl.BlockSpec((G_BLK, D),
                               lambda b, bl, na: (beff(b, bl, na), 0)),
    )
    return pl.pallas_call(
        _group_body,
        grid_spec=grid_spec,
        out_shape=jax.ShapeDtypeStruct((NSLOT, D), F32),
        compiler_params=pltpu.CompilerParams(
            dimension_semantics=("arbitrary",)),
    )(blk, na, hperm, wperm, we1, be1.reshape(E, 1, DH), we2,
      be2.reshape(E, 1, D))


def _combine_body(o1_hbm, yp_hbm, s0_hbm, s1_hbm, out_hbm,
                  obuf, y0buf, y1buf, s0v, s1v, sem0, sem1):
    wid = lax.axis_index("s") * 2 + lax.axis_index("c")
    base = wid * TPW
    half = TPW // 2

    for hf in range(2):
        r0 = base + hf * half
        pltpu.sync_copy(o1_hbm.at[pl.ds(r0, half)], obuf)
        pltpu.sync_copy(s0_hbm.at[pl.ds(r0, half)], s0v)
        pltpu.sync_copy(s1_hbm.at[pl.ds(r0, half)], s1v)
        g0 = pltpu.async_copy(yp_hbm.at[s0v], y0buf, sem0)
        g1 = pltpu.async_copy(yp_hbm.at[s1v], y1buf, sem1)
        g0.wait()
        g1.wait()

        def row_body(i, _):
            for c in range(D // 16):
                sl = pl.ds(c * 16, 16)
                obuf[i, sl] = obuf[i, sl] + y0buf[i, sl] + y1buf[i, sl]
            return 0

        lax.fori_loop(0, half, row_body, 0)
        pltpu.sync_copy(obuf, out_hbm.at[pl.ds(r0, half)])


def _stage_e(out1, yperm, s0, s1):
    mesh = plsc.VectorSubcoreMesh(core_axis_name="c", subcore_axis_name="s", num_cores=2, num_subcores=16)
    half = TPW // 2
    f = pl.kernel(
        _combine_body,
        out_type=jax.ShapeDtypeStruct((S, D), F32),
        mesh=mesh,
        compiler_params=pltpu.CompilerParams(needs_layout_passes=False),
        scratch_types=[
            pltpu.VMEM((half, D), F32), pltpu.VMEM((half, D), F32),
            pltpu.VMEM((half, D), F32),
            pltpu.VMEM((half,), I32), pltpu.VMEM((half,), I32),
            pltpu.SemaphoreType.DMA, pltpu.SemaphoreType.DMA,
        ],
    )
    return f(out1, yperm, s0, s1)


@jax.jit
def kernel(hidden_states, ln_g, ln_b, W_fc1, b_fc1, W_fc2, b_fc2,
           W_router, b_router, We1, be1, We2, be2):
    x = hidden_states.reshape(S, D)
    (h, wr0, wr1, e0o, e1o, r0o, r1o, cnts) = _stage_a(
        x, ln_g, ln_b, W_router, b_router)

    s0o, s1o, blk, na = _stage_a2(cnts, e0o, e1o, r0o, r1o)
    s0 = s0o.reshape(S)
    s1 = s1o.reshape(S)
    hperm, wperm = _stage_c(h, wr0, wr1, s0, s1)
    out1 = _stage_b(x, h, W_fc1, b_fc1, W_fc2, b_fc2)
    yperm = _stage_d(blk.reshape(NW), na.reshape(16), hperm, wperm,
                     We1, be1, We2, be2)
    out = _stage_e(out1, yperm, s0, s1)
    return out.reshape(B, S, D)


# speedup vs baseline: 1.2874x; 1.0436x over previous
import jax
import jax.numpy as jnp
from jax import lax
from jax.experimental import pallas as pl
from jax.experimental.pallas import tpu as pltpu
from jax.experimental.pallas import tpu_sc as plsc

B, S, D = 1, 2048, 768
DFF = 3072
E = 8
DH = 768
TEMP = 1.0
EPS = 1e-5

T_BLK = 256
NB = S // T_BLK
G_BLK = 256
NSLOT = 2 * S + E * G_BLK
ND = NSLOT // G_BLK
NW = 32
TPW = S // NW
BF = jnp.bfloat16
F32 = jnp.float32
I32 = jnp.int32


def _route_body(x_ref, lng_ref, lnb_ref, wr_ref, br_ref,
                h_ref, w1_ref, w2_ref, e0_ref, e1_ref, r0_ref, r1_ref,
                cnt_ref, run_ref):
    @pl.when(pl.program_id(0) == 0)
    def _():
        run_ref[...] = jnp.zeros((1, 16), F32)

    xb = x_ref[...]
    mu = jnp.mean(xb, axis=-1, keepdims=True)
    var = jnp.mean((xb - mu) ** 2, axis=-1, keepdims=True)
    hb = (xb - mu) / jnp.sqrt(var + EPS) * lng_ref[...] + lnb_ref[...]
    h_ref[...] = hb

    logits = jnp.dot(hb.astype(BF), wr_ref[...].astype(BF),
                     preferred_element_type=F32) + br_ref[...]
    idx = lax.broadcasted_iota(I32, (T_BLK, E), 1)
    m1 = jnp.max(logits, axis=-1, keepdims=True)
    i1 = jnp.min(jnp.where(logits == m1, idx, E), axis=-1, keepdims=True)
    mask1 = idx == i1
    rest = jnp.where(mask1, -jnp.inf, logits)
    m2 = jnp.max(rest, axis=-1, keepdims=True)
    i2 = jnp.min(jnp.where(rest == m2, idx, E), axis=-1, keepdims=True)
    mask2 = idx == i2
    w1 = 1.0 / (1.0 + jnp.exp((m2 - m1) / TEMP))

    m1f = mask1.astype(F32)
    m2f = mask2.astype(F32)
    mm = m1f + m2f
    ri = lax.broadcasted_iota(I32, (T_BLK, T_BLK), 0)
    ci = lax.broadcasted_iota(I32, (T_BLK, T_BLK), 1)
    ltri = (ci < ri).astype(BF)
    csum = jnp.dot(ltri, mm.astype(BF), preferred_element_type=F32)
    run = run_ref[...][:, :E]
    cg = csum + run
    r0 = jnp.sum(jnp.where(mask1, cg, 0.0), axis=-1)
    r1 = jnp.sum(jnp.where(mask2, cg + m1f, 0.0), axis=-1)
    newrun = run + jnp.sum(mm, axis=0, keepdims=True)
    run_ref[:, :E] = newrun
    cnt_ref[:, :E] = newrun.astype(I32)
    cnt_ref[:, E:] = jnp.zeros((1, 16 - E), I32)

    ones128 = jnp.ones((1, 128), F32)
    w1_ref[...] = w1 * ones128
    w2_ref[...] = (1.0 - w1) * ones128
    e0_ref[...] = i1.astype(I32).reshape(1, 1, T_BLK)
    e1_ref[...] = i2.astype(I32).reshape(1, 1, T_BLK)
    r0_ref[...] = r0.astype(I32).reshape(1, 1, T_BLK)
    r1_ref[...] = r1.astype(I32).reshape(1, 1, T_BLK)


def _stage_a(x, ln_g, ln_b, wr, br):
    full = lambda shape: pl.BlockSpec(shape, lambda t: (0,) * len(shape))
    per_tok = lambda dt: jax.ShapeDtypeStruct((NB, 1, T_BLK), dt)
    return pl.pallas_call(
        _route_body,
        grid=(NB,),
        in_specs=[
            pl.BlockSpec((T_BLK, D), lambda t: (t, 0)),
            full((1, D)), full((1, D)), full((D, E)), full((1, E)),
        ],
        out_specs=[
            pl.BlockSpec((T_BLK, D), lambda t: (t, 0)),
            pl.BlockSpec((T_BLK, 128), lambda t: (t, 0)),
            pl.BlockSpec((T_BLK, 128), lambda t: (t, 0)),
            pl.BlockSpec((1, 1, T_BLK), lambda t: (t, 0, 0)),
            pl.BlockSpec((1, 1, T_BLK), lambda t: (t, 0, 0)),
            pl.BlockSpec((1, 1, T_BLK), lambda t: (t, 0, 0)),
            pl.BlockSpec((1, 1, T_BLK), lambda t: (t, 0, 0)),
            full((1, 16)),
        ],
        out_shape=[
            jax.ShapeDtypeStruct((S, D), F32),
            jax.ShapeDtypeStruct((S, 128), F32),
            jax.ShapeDtypeStruct((S, 128), F32),
            per_tok(I32), per_tok(I32), per_tok(I32), per_tok(I32),
            jax.ShapeDtypeStruct((1, 16), I32),
        ],
        scratch_shapes=[pltpu.VMEM((1, 16), F32)],
        compiler_params=pltpu.CompilerParams(
            dimension_semantics=("arbitrary",)),
    )(x, ln_g.reshape(1, D), ln_b.reshape(1, D), wr, br.reshape(1, E))


def _shared_body(x_ref, h_ref, w1_ref, b1_ref, w2_ref, b2_ref, o_ref):
    hbb = h_ref[...].astype(BF)
    t1 = jnp.dot(hbb, w1_ref[...].astype(BF),
                 preferred_element_type=F32) + b1_ref[...]
    a1 = jax.nn.gelu(t1.astype(BF))
    sh = jnp.dot(a1, w2_ref[...].astype(BF),
                 preferred_element_type=F32) + b2_ref[...]
    o_ref[...] = x_ref[...] + sh


def _stage_b(x, h, wfc1, bfc1, wfc2, bfc2):
    full = lambda shape: pl.BlockSpec(shape, lambda t: (0,) * len(shape))
    return pl.pallas_call(
        _shared_body,
        grid=(NB,),
        in_specs=[
            pl.BlockSpec((T_BLK, D), lambda t: (t, 0)),
            pl.BlockSpec((T_BLK, D), lambda t: (t, 0)),
            full((D, DFF)), full((1, DFF)), full((DFF, D)), full((1, D)),
        ],
        out_specs=pl.BlockSpec((T_BLK, D), lambda t: (t, 0)),
        out_shape=jax.ShapeDtypeStruct((S, D), F32),
        compiler_params=pltpu.CompilerParams(
            dimension_semantics=("arbitrary",)),
    )(x, h, wfc1, bfc1.reshape(1, DFF), wfc2, bfc2.reshape(1, D))


def _slots_body(cnt_ref, e0_ref, e1_ref, r0_ref, r1_ref,
                s0_ref, s1_ref, blk_ref, na_ref):
    cntb = cnt_ref[...][:, :E].astype(F32)
    pb = jnp.floor((cntb + (G_BLK - 1)) * (1.0 / G_BLK))
    ri = lax.broadcasted_iota(I32, (E, E), 0)
    ci = lax.broadcasted_iota(I32, (E, E), 1)
    utri = (ri < ci).astype(BF)
    baseb = jnp.dot(pb.astype(BF), utri, preferred_element_type=F32)

    e0 = e0_ref[...].reshape(1, T_BLK)
    e1 = e1_ref[...].reshape(1, T_BLK)
    s0 = r0_ref[...].reshape(1, T_BLK).astype(F32)
    s1 = r1_ref[...].reshape(1, T_BLK).astype(F32)
    iota8 = lax.broadcasted_iota(I32, (1, E), 1)
    bstart = lax.broadcasted_iota(I32, (1, NW), 1).astype(F32)
    blk = jnp.zeros((1, NW), F32)
    na = jnp.zeros((1, 16), F32)
    for e in range(E):
        be = jnp.sum(jnp.where(iota8 == e, baseb, 0.0), axis=-1,
                     keepdims=True)
        pe = jnp.sum(jnp.where(iota8 == e, pb, 0.0), axis=-1, keepdims=True)
        s0 = s0 + jnp.where(e0 == e, be * G_BLK, 0.0)
        s1 = s1 + jnp.where(e1 == e, be * G_BLK, 0.0)
        blk = blk + jnp.where(bstart >= be, 1.0, 0.0)
        na = na + pe
    s0_ref[...] = s0.astype(I32).reshape(1, 1, T_BLK)
    s1_ref[...] = s1.astype(I32).reshape(1, 1, T_BLK)
    blk_ref[...] = (blk - 1.0).astype(I32)
    na_ref[...] = na.astype(I32)


def _stage_a2(cnts, e0o, e1o, r0o, r1o):
    full = lambda shape: pl.BlockSpec(shape, lambda t: (0,) * len(shape))
    tok = pl.BlockSpec((1, 1, T_BLK), lambda t: (t, 0, 0))
    return pl.pallas_call(
        _slots_body,
        grid=(NB,),
        in_specs=[full((1, 16)), tok, tok, tok, tok],
        out_specs=[tok, tok, full((1, NW)), full((1, 16))],
        out_shape=[
            jax.ShapeDtypeStruct((NB, 1, T_BLK), I32),
            jax.ShapeDtypeStruct((NB, 1, T_BLK), I32),
            jax.ShapeDtypeStruct((1, NW), I32),
            jax.ShapeDtypeStruct((1, 16), I32),
        ],
        compiler_params=pltpu.CompilerParams(
            dimension_semantics=("arbitrary",)),
    )(cnts, e0o, e1o, r0o, r1o)


def _dispatch_body(h_hbm, wr0_hbm, wr1_hbm, s0_hbm, s1_hbm,
                   hperm_hbm, wperm_hbm,
                   s0v, s1v, hrows, w0rows, w1rows,
                   sem0, sem1, sem2, sem3):
    wid = lax.axis_index("s") * 2 + lax.axis_index("c")
    base = wid * TPW
    pltpu.sync_copy(s0_hbm.at[pl.ds(base, TPW)], s0v)
    pltpu.sync_copy(s1_hbm.at[pl.ds(base, TPW)], s1v)
    pltpu.sync_copy(h_hbm.at[pl.ds(base, TPW)], hrows)
    pltpu.sync_copy(wr0_hbm.at[pl.ds(base, TPW)], w0rows)
    pltpu.sync_copy(wr1_hbm.at[pl.ds(base, TPW)], w1rows)
    d0 = pltpu.async_copy(hrows, hperm_hbm.at[s0v], sem0)
    d1 = pltpu.async_copy(hrows, hperm_hbm.at[s1v], sem1)
    d2 = pltpu.async_copy(w0rows, wperm_hbm.at[s0v], sem2)
    d3 = pltpu.async_copy(w1rows, wperm_hbm.at[s1v], sem3)
    d0.wait()
    d1.wait()
    d2.wait()
    d3.wait()


def _stage_c(h, wr0, wr1, s0, s1):
    mesh = plsc.VectorSubcoreMesh(core_axis_name="c", subcore_axis_name="s", num_cores=2, num_subcores=16)
    f = pl.kernel(
        _dispatch_body,
        out_type=[
            jax.ShapeDtypeStruct((NSLOT, D), F32),
            jax.ShapeDtypeStruct((NSLOT, 128), F32),
        ],
        mesh=mesh,
        compiler_params=pltpu.CompilerParams(needs_layout_passes=False),
        scratch_types=[
            pltpu.VMEM((TPW,), I32), pltpu.VMEM((TPW,), I32),
            pltpu.VMEM((TPW, D), F32),
            pltpu.VMEM((TPW, 128), F32), pltpu.VMEM((TPW, 128), F32),
            pltpu.SemaphoreType.DMA, pltpu.SemaphoreType.DMA,
            pltpu.SemaphoreType.DMA, pltpu.SemaphoreType.DMA,
        ],
    )
    return f(h, wr0, wr1, s0, s1)


def _group_body(blk_sref, na_sref, hp_ref, wp_ref, w1_ref, b1_ref, w2_ref,
                b2_ref, y_ref):
    b = pl.program_id(0)

    @pl.when(b < na_sref[0])
    def _():
        xb = hp_ref[...].astype(BF)
        t1 = jnp.dot(xb, w1_ref[0].astype(BF),
                     preferred_element_type=F32) + b1_ref[0, 0]
        a1 = jax.nn.gelu(t1.astype(BF))
        y = jnp.dot(a1, w2_ref[0].astype(BF),
                    preferred_element_type=F32) + b2_ref[0, 0]
        y_ref[...] = y * wp_ref[...][:, 0:1]


def _stage_d(blk, na, hperm, wperm, we1, be1, we2, be2):
    def beff(b, blk_r, na_r):
        return jnp.minimum(b, na_r[0] - 1)

    grid_spec = pltpu.PrefetchScalarGridSpec(
        num_scalar_prefetch=2,
        grid=(ND,),
        in_specs=[
            pl.BlockSpec((G_BLK, D), lambda b, bl, na: (beff(b, bl, na), 0)),
            pl.BlockSpec((G_BLK, 128),
                         lambda b, bl, na: (beff(b, bl, na), 0)),
            pl.BlockSpec((1, D, DH),
                         lambda b, bl, na: (bl[beff(b, bl, na)], 0, 0)),
            pl.BlockSpec((1, 1, DH),
                         lambda b, bl, na: (bl[beff(b, bl, na)], 0, 0)),
            pl.BlockSpec((1, DH, D),
                         lambda b, bl, na: (bl[beff(b, bl, na)], 0, 0)),
            pl.BlockSpec((1, 1, D),
                         lambda b, bl, na: (bl[beff(b, bl, na)], 0, 0)),
        ],
        out_specs=pl.BlockSpec((G_BLK, D),
                               lambda b, bl, na: (beff(b, bl, na), 0)),
    )
    return pl.pallas_call(
        _group_body,
        grid_spec=grid_spec,
        out_shape=jax.ShapeDtypeStruct((NSLOT, D), F32),
        compiler_params=pltpu.CompilerParams(
            dimension_semantics=("arbitrary",)),
    )(blk, na, hperm, wperm, we1, be1.reshape(E, 1, DH), we2,
      be2.reshape(E, 1, D))


_EQ = 16
_ENQ = TPW // _EQ


def _combine_body(o1_hbm, yp_hbm, s0_hbm, s1_hbm, out_hbm,
                  s0v, s1v, ob0, ob1, y00, y01, y10, y11,
                  so0, so1, sg00, sg01, sg10, sg11):
    wid = lax.axis_index("s") * 2 + lax.axis_index("c")
    base = wid * TPW
    obufs, y0s, y1s = [ob0, ob1], [y00, y01], [y10, y11]
    semo, sem0s, sem1s = [so0, so1], [sg00, sg01], [sg10, sg11]

    pltpu.sync_copy(s0_hbm.at[pl.ds(base, TPW)], s0v)
    pltpu.sync_copy(s1_hbm.at[pl.ds(base, TPW)], s1v)

    def issue(q):
        i = q % 2
        qb = base + q * _EQ
        do = pltpu.async_copy(o1_hbm.at[pl.ds(qb, _EQ)], obufs[i], semo[i])
        d0 = pltpu.async_copy(yp_hbm.at[s0v.at[pl.ds(q * _EQ, _EQ)]],
                              y0s[i], sem0s[i])
        d1 = pltpu.async_copy(yp_hbm.at[s1v.at[pl.ds(q * _EQ, _EQ)]],
                              y1s[i], sem1s[i])
        return (do, d0, d1)

    pending = issue(0)
    for q in range(_ENQ):
        nxt = issue(q + 1) if q + 1 < _ENQ else None
        for dsc in pending:
            dsc.wait()
        i = q % 2
        buf, yy0, yy1 = obufs[i], y0s[i], y1s[i]

        def row_body(r, _):
            for c in range(D // 16):
                sl = pl.ds(c * 16, 16)
                buf[r, sl] = buf[r, sl] + yy0[r, sl] + yy1[r, sl]
            return 0

        lax.fori_loop(0, _EQ, row_body, 0)
        pltpu.sync_copy(buf, out_hbm.at[pl.ds(base + q * _EQ, _EQ)])
        pending = nxt


def _stage_e(out1, yperm, s0, s1):
    mesh = plsc.VectorSubcoreMesh(core_axis_name="c", subcore_axis_name="s", num_cores=2, num_subcores=16)
    f = pl.kernel(
        _combine_body,
        out_type=jax.ShapeDtypeStruct((S, D), F32),
        mesh=mesh,
        compiler_params=pltpu.CompilerParams(needs_layout_passes=False),
        scratch_types=[
            pltpu.VMEM((TPW,), I32), pltpu.VMEM((TPW,), I32),
            pltpu.VMEM((_EQ, D), F32), pltpu.VMEM((_EQ, D), F32),
            pltpu.VMEM((_EQ, D), F32), pltpu.VMEM((_EQ, D), F32),
            pltpu.VMEM((_EQ, D), F32), pltpu.VMEM((_EQ, D), F32),
            pltpu.SemaphoreType.DMA, pltpu.SemaphoreType.DMA,
            pltpu.SemaphoreType.DMA, pltpu.SemaphoreType.DMA,
            pltpu.SemaphoreType.DMA, pltpu.SemaphoreType.DMA,
        ],
    )
    return f(out1, yperm, s0, s1)


@jax.jit
def kernel(hidden_states, ln_g, ln_b, W_fc1, b_fc1, W_fc2, b_fc2,
           W_router, b_router, We1, be1, We2, be2):
    x = hidden_states.reshape(S, D)
    (h, wr0, wr1, e0o, e1o, r0o, r1o, cnts) = _stage_a(
        x, ln_g, ln_b, W_router, b_router)

    s0o, s1o, blk, na = _stage_a2(cnts, e0o, e1o, r0o, r1o)
    s0 = s0o.reshape(S)
    s1 = s1o.reshape(S)
    hperm, wperm = _stage_c(h, wr0, wr1, s0, s1)
    out1 = _stage_b(x, h, W_fc1, b_fc1, W_fc2, b_fc2)
    yperm = _stage_d(blk.reshape(NW), na.reshape(16), hperm, wperm,
                     We1, be1, We2, be2)
    out = _stage_e(out1, yperm, s0, s1)
    return out.reshape(B, S, D)


# speedup vs baseline: 1.3509x; 1.0493x over previous
import jax
import jax.numpy as jnp
from jax import lax
from jax.experimental import pallas as pl
from jax.experimental.pallas import tpu as pltpu
from jax.experimental.pallas import tpu_sc as plsc

B, S, D = 1, 2048, 768
DFF = 3072
E = 8
DH = 768
TEMP = 1.0
EPS = 1e-5

T_BLK = 256
NB = S // T_BLK
G_BLK = 512
NSLOT = 2 * S + E * G_BLK
ND = NSLOT // G_BLK
NW = 32
TPW = S // NW
BF = jnp.bfloat16
F32 = jnp.float32
I32 = jnp.int32


def _route_body(x_ref, lng_ref, lnb_ref, wr_ref, br_ref,
                h_ref, w1_ref, w2_ref, e0_ref, e1_ref, r0_ref, r1_ref,
                cnt_ref, run_ref):
    @pl.when(pl.program_id(0) == 0)
    def _():
        run_ref[...] = jnp.zeros((1, 16), F32)

    xb = x_ref[...]
    mu = jnp.mean(xb, axis=-1, keepdims=True)
    var = jnp.mean((xb - mu) ** 2, axis=-1, keepdims=True)
    hb = (xb - mu) / jnp.sqrt(var + EPS) * lng_ref[...] + lnb_ref[...]
    h_ref[...] = hb

    logits = jnp.dot(hb.astype(BF), wr_ref[...].astype(BF),
                     preferred_element_type=F32) + br_ref[...]
    idx = lax.broadcasted_iota(I32, (T_BLK, E), 1)
    m1 = jnp.max(logits, axis=-1, keepdims=True)
    i1 = jnp.min(jnp.where(logits == m1, idx, E), axis=-1, keepdims=True)
    mask1 = idx == i1
    rest = jnp.where(mask1, -jnp.inf, logits)
    m2 = jnp.max(rest, axis=-1, keepdims=True)
    i2 = jnp.min(jnp.where(rest == m2, idx, E), axis=-1, keepdims=True)
    mask2 = idx == i2
    w1 = 1.0 / (1.0 + jnp.exp((m2 - m1) / TEMP))

    m1f = mask1.astype(F32)
    m2f = mask2.astype(F32)
    mm = m1f + m2f
    ri = lax.broadcasted_iota(I32, (T_BLK, T_BLK), 0)
    ci = lax.broadcasted_iota(I32, (T_BLK, T_BLK), 1)
    ltri = (ci < ri).astype(BF)
    csum = jnp.dot(ltri, mm.astype(BF), preferred_element_type=F32)
    run = run_ref[...][:, :E]
    cg = csum + run
    r0 = jnp.sum(jnp.where(mask1, cg, 0.0), axis=-1)
    r1 = jnp.sum(jnp.where(mask2, cg + m1f, 0.0), axis=-1)
    newrun = run + jnp.sum(mm, axis=0, keepdims=True)
    run_ref[:, :E] = newrun
    cnt_ref[:, :E] = newrun.astype(I32)
    cnt_ref[:, E:] = jnp.zeros((1, 16 - E), I32)

    ones128 = jnp.ones((1, 128), F32)
    w1_ref[...] = w1 * ones128
    w2_ref[...] = (1.0 - w1) * ones128
    e0_ref[...] = i1.astype(I32).reshape(1, 1, T_BLK)
    e1_ref[...] = i2.astype(I32).reshape(1, 1, T_BLK)
    r0_ref[...] = r0.astype(I32).reshape(1, 1, T_BLK)
    r1_ref[...] = r1.astype(I32).reshape(1, 1, T_BLK)


def _stage_a(x, ln_g, ln_b, wr, br):
    full = lambda shape: pl.BlockSpec(shape, lambda t: (0,) * len(shape))
    per_tok = lambda dt: jax.ShapeDtypeStruct((NB, 1, T_BLK), dt)
    return pl.pallas_call(
        _route_body,
        grid=(NB,),
        in_specs=[
            pl.BlockSpec((T_BLK, D), lambda t: (t, 0)),
            full((1, D)), full((1, D)), full((D, E)), full((1, E)),
        ],
        out_specs=[
            pl.BlockSpec((T_BLK, D), lambda t: (t, 0)),
            pl.BlockSpec((T_BLK, 128), lambda t: (t, 0)),
            pl.BlockSpec((T_BLK, 128), lambda t: (t, 0)),
            pl.BlockSpec((1, 1, T_BLK), lambda t: (t, 0, 0)),
            pl.BlockSpec((1, 1, T_BLK), lambda t: (t, 0, 0)),
            pl.BlockSpec((1, 1, T_BLK), lambda t: (t, 0, 0)),
            pl.BlockSpec((1, 1, T_BLK), lambda t: (t, 0, 0)),
            full((1, 16)),
        ],
        out_shape=[
            jax.ShapeDtypeStruct((S, D), F32),
            jax.ShapeDtypeStruct((S, 128), F32),
            jax.ShapeDtypeStruct((S, 128), F32),
            per_tok(I32), per_tok(I32), per_tok(I32), per_tok(I32),
            jax.ShapeDtypeStruct((1, 16), I32),
        ],
        scratch_shapes=[pltpu.VMEM((1, 16), F32)],
        compiler_params=pltpu.CompilerParams(
            dimension_semantics=("arbitrary",)),
    )(x, ln_g.reshape(1, D), ln_b.reshape(1, D), wr, br.reshape(1, E))


def _shared_body(x_ref, h_ref, w1_ref, b1_ref, w2_ref, b2_ref, o_ref):
    hbb = h_ref[...].astype(BF)
    t1 = jnp.dot(hbb, w1_ref[...].astype(BF),
                 preferred_element_type=F32) + b1_ref[...]
    a1 = jax.nn.gelu(t1.astype(BF))
    sh = jnp.dot(a1, w2_ref[...].astype(BF),
                 preferred_element_type=F32) + b2_ref[...]
    o_ref[...] = x_ref[...] + sh


def _stage_b(x, h, wfc1, bfc1, wfc2, bfc2):
    full = lambda shape: pl.BlockSpec(shape, lambda t: (0,) * len(shape))
    return pl.pallas_call(
        _shared_body,
        grid=(NB,),
        in_specs=[
            pl.BlockSpec((T_BLK, D), lambda t: (t, 0)),
            pl.BlockSpec((T_BLK, D), lambda t: (t, 0)),
            full((D, DFF)), full((1, DFF)), full((DFF, D)), full((1, D)),
        ],
        out_specs=pl.BlockSpec((T_BLK, D), lambda t: (t, 0)),
        out_shape=jax.ShapeDtypeStruct((S, D), F32),
        compiler_params=pltpu.CompilerParams(
            dimension_semantics=("arbitrary",)),
    )(x, h, wfc1, bfc1.reshape(1, DFF), wfc2, bfc2.reshape(1, D))


def _slots_body(cnt_ref, e0_ref, e1_ref, r0_ref, r1_ref,
                s0_ref, s1_ref, blk_ref, na_ref):
    cntb = cnt_ref[...][:, :E].astype(F32)
    pb = jnp.floor((cntb + (G_BLK - 1)) * (1.0 / G_BLK))
    ri = lax.broadcasted_iota(I32, (E, E), 0)
    ci = lax.broadcasted_iota(I32, (E, E), 1)
    utri = (ri < ci).astype(BF)
    baseb = jnp.dot(pb.astype(BF), utri, preferred_element_type=F32)

    e0 = e0_ref[...].reshape(1, T_BLK)
    e1 = e1_ref[...].reshape(1, T_BLK)
    s0 = r0_ref[...].reshape(1, T_BLK).astype(F32)
    s1 = r1_ref[...].reshape(1, T_BLK).astype(F32)
    iota8 = lax.broadcasted_iota(I32, (1, E), 1)
    bstart = lax.broadcasted_iota(I32, (1, NW), 1).astype(F32)
    blk = jnp.zeros((1, NW), F32)
    na = jnp.zeros((1, 16), F32)
    for e in range(E):
        be = jnp.sum(jnp.where(iota8 == e, baseb, 0.0), axis=-1,
                     keepdims=True)
        pe = jnp.sum(jnp.where(iota8 == e, pb, 0.0), axis=-1, keepdims=True)
        s0 = s0 + jnp.where(e0 == e, be * G_BLK, 0.0)
        s1 = s1 + jnp.where(e1 == e, be * G_BLK, 0.0)
        blk = blk + jnp.where(bstart >= be, 1.0, 0.0)
        na = na + pe
    s0_ref[...] = s0.astype(I32).reshape(1, 1, T_BLK)
    s1_ref[...] = s1.astype(I32).reshape(1, 1, T_BLK)
    blk_ref[...] = (blk - 1.0).astype(I32)
    na_ref[...] = na.astype(I32)


def _stage_a2(cnts, e0o, e1o, r0o, r1o):
    full = lambda shape: pl.BlockSpec(shape, lambda t: (0,) * len(shape))
    tok = pl.BlockSpec((1, 1, T_BLK), lambda t: (t, 0, 0))
    return pl.pallas_call(
        _slots_body,
        grid=(NB,),
        in_specs=[full((1, 16)), tok, tok, tok, tok],
        out_specs=[tok, tok, full((1, NW)), full((1, 16))],
        out_shape=[
            jax.ShapeDtypeStruct((NB, 1, T_BLK), I32),
            jax.ShapeDtypeStruct((NB, 1, T_BLK), I32),
            jax.ShapeDtypeStruct((1, NW), I32),
            jax.ShapeDtypeStruct((1, 16), I32),
        ],
        compiler_params=pltpu.CompilerParams(
            dimension_semantics=("arbitrary",)),
    )(cnts, e0o, e1o, r0o, r1o)


def _dispatch_body(h_hbm, wr0_hbm, wr1_hbm, s0_hbm, s1_hbm,
                   hperm_hbm, wperm_hbm,
                   s0v, s1v, hrows, w0rows, w1rows,
                   sem0, sem1, sem2, sem3):
    wid = lax.axis_index("s") * 2 + lax.axis_index("c")
    base = wid * TPW
    pltpu.sync_copy(s0_hbm.at[pl.ds(base, TPW)], s0v)
    pltpu.sync_copy(s1_hbm.at[pl.ds(base, TPW)], s1v)
    pltpu.sync_copy(h_hbm.at[pl.ds(base, TPW)], hrows)
    pltpu.sync_copy(wr0_hbm.at[pl.ds(base, TPW)], w0rows)
    pltpu.sync_copy(wr1_hbm.at[pl.ds(base, TPW)], w1rows)
    d0 = pltpu.async_copy(hrows, hperm_hbm.at[s0v], sem0)
    d1 = pltpu.async_copy(hrows, hperm_hbm.at[s1v], sem1)
    d2 = pltpu.async_copy(w0rows, wperm_hbm.at[s0v], sem2)
    d3 = pltpu.async_copy(w1rows, wperm_hbm.at[s1v], sem3)
    d0.wait()
    d1.wait()
    d2.wait()
    d3.wait()


def _stage_c(h, wr0, wr1, s0, s1):
    mesh = plsc.VectorSubcoreMesh(core_axis_name="c", subcore_axis_name="s", num_cores=2, num_subcores=16)
    f = pl.kernel(
        _dispatch_body,
        out_type=[
            jax.ShapeDtypeStruct((NSLOT, D), F32),
            jax.ShapeDtypeStruct((NSLOT, 128), F32),
        ],
        mesh=mesh,
        compiler_params=pltpu.CompilerParams(needs_layout_passes=False),
        scratch_types=[
            pltpu.VMEM((TPW,), I32), pltpu.VMEM((TPW,), I32),
            pltpu.VMEM((TPW, D), F32),
            pltpu.VMEM((TPW, 128), F32), pltpu.VMEM((TPW, 128), F32),
            pltpu.SemaphoreType.DMA, pltpu.SemaphoreType.DMA,
            pltpu.SemaphoreType.DMA, pltpu.SemaphoreType.DMA,
        ],
    )
    return f(h, wr0, wr1, s0, s1)


def _group_body(blk_sref, na_sref, hp_ref, wp_ref, w1_ref, b1_ref, w2_ref,
                b2_ref, y_ref):
    b = pl.program_id(0)

    @pl.when(b < na_sref[0])
    def _():
        xb = hp_ref[...].astype(BF)
        t1 = jnp.dot(xb, w1_ref[0].astype(BF),
                     preferred_element_type=F32) + b1_ref[0, 0]
        a1 = jax.nn.gelu(t1.astype(BF))
        y = jnp.dot(a1, w2_ref[0].astype(BF),
                    preferred_element_type=F32) + b2_ref[0, 0]
        y_ref[...] = y * wp_ref[...][:, 0:1]


def _stage_d(blk, na, hperm, wperm, we1, be1, we2, be2):
    def beff(b, blk_r, na_r):
        return jnp.minimum(b, na_r[0] - 1)

    grid_spec = pltpu.PrefetchScalarGridSpec(
        num_scalar_prefetch=2,
        grid=(ND,),
        in_specs=[
            pl.BlockSpec((G_BLK, D), lambda b, bl, na: (beff(b, bl, na), 0)),
            pl.BlockSpec((G_BLK, 128),
                         lambda b, bl, na: (beff(b, bl, na), 0)),
            pl.BlockSpec((1, D, DH),
                         lambda b, bl, na: (bl[beff(b, bl, na)], 0, 0)),
            pl.BlockSpec((1, 1, DH),
                         lambda b, bl, na: (bl[beff(b, bl, na)], 0, 0)),
            pl.BlockSpec((1, DH, D),
                         lambda b, bl, na: (bl[beff(b, bl, na)], 0, 0)),
            pl.BlockSpec((1, 1, D),
                         lambda b, bl, na: (bl[beff(b, bl, na)], 0, 0)),
        ],
        out_specs=pl.BlockSpec((G_BLK, D),
                               lambda b, bl, na: (beff(b, bl, na), 0)),
    )
    return pl.pallas_call(
        _group_body,
        grid_spec=grid_spec,
        out_shape=jax.ShapeDtypeStruct((NSLOT, D), F32),
        compiler_params=pltpu.CompilerParams(
            dimension_semantics=("arbitrary",)),
    )(blk, na, hperm, wperm, we1, be1.reshape(E, 1, DH), we2,
      be2.reshape(E, 1, D))


_EQ = 16
_ENQ = TPW // _EQ


def _combine_body(o1_hbm, yp_hbm, s0_hbm, s1_hbm, out_hbm,
                  s0v, s1v, ob0, ob1, y00, y01, y10, y11,
                  so0, so1, sg00, sg01, sg10, sg11):
    wid = lax.axis_index("s") * 2 + lax.axis_index("c")
    base = wid * TPW
    obufs, y0s, y1s = [ob0, ob1], [y00, y01], [y10, y11]
    semo, sem0s, sem1s = [so0, so1], [sg00, sg01], [sg10, sg11]

    pltpu.sync_copy(s0_hbm.at[pl.ds(base, TPW)], s0v)
    pltpu.sync_copy(s1_hbm.at[pl.ds(base, TPW)], s1v)

    def issue(q):
        i = q % 2
        qb = base + q * _EQ
        do = pltpu.async_copy(o1_hbm.at[pl.ds(qb, _EQ)], obufs[i], semo[i])
        d0 = pltpu.async_copy(yp_hbm.at[s0v.at[pl.ds(q * _EQ, _EQ)]],
                              y0s[i], sem0s[i])
        d1 = pltpu.async_copy(yp_hbm.at[s1v.at[pl.ds(q * _EQ, _EQ)]],
                              y1s[i], sem1s[i])
        return (do, d0, d1)

    pending = issue(0)
    for q in range(_ENQ):
        nxt = issue(q + 1) if q + 1 < _ENQ else None
        for dsc in pending:
            dsc.wait()
        i = q % 2
        buf, yy0, yy1 = obufs[i], y0s[i], y1s[i]

        def row_body(r, _):
            for c in range(D // 16):
                sl = pl.ds(c * 16, 16)
                buf[r, sl] = buf[r, sl] + yy0[r, sl] + yy1[r, sl]
            return 0

        lax.fori_loop(0, _EQ, row_body, 0)
        pltpu.sync_copy(buf, out_hbm.at[pl.ds(base + q * _EQ, _EQ)])
        pending = nxt


def _stage_e(out1, yperm, s0, s1):
    mesh = plsc.VectorSubcoreMesh(core_axis_name="c", subcore_axis_name="s", num_cores=2, num_subcores=16)
    f = pl.kernel(
        _combine_body,
        out_type=jax.ShapeDtypeStruct((S, D), F32),
        mesh=mesh,
        compiler_params=pltpu.CompilerParams(needs_layout_passes=False),
        scratch_types=[
            pltpu.VMEM((TPW,), I32), pltpu.VMEM((TPW,), I32),
            pltpu.VMEM((_EQ, D), F32), pltpu.VMEM((_EQ, D), F32),
            pltpu.VMEM((_EQ, D), F32), pltpu.VMEM((_EQ, D), F32),
            pltpu.VMEM((_EQ, D), F32), pltpu.VMEM((_EQ, D), F32),
            pltpu.SemaphoreType.DMA, pltpu.SemaphoreType.DMA,
            pltpu.SemaphoreType.DMA, pltpu.SemaphoreType.DMA,
            pltpu.SemaphoreType.DMA, pltpu.SemaphoreType.DMA,
        ],
    )
    return f(out1, yperm, s0, s1)


@jax.jit
def kernel(hidden_states, ln_g, ln_b, W_fc1, b_fc1, W_fc2, b_fc2,
           W_router, b_router, We1, be1, We2, be2):
    x = hidden_states.reshape(S, D)
    (h, wr0, wr1, e0o, e1o, r0o, r1o, cnts) = _stage_a(
        x, ln_g, ln_b, W_router, b_router)

    s0o, s1o, blk, na = _stage_a2(cnts, e0o, e1o, r0o, r1o)
    s0 = s0o.reshape(S)
    s1 = s1o.reshape(S)
    hperm, wperm = _stage_c(h, wr0, wr1, s0, s1)
    out1 = _stage_b(x, h, W_fc1, b_fc1, W_fc2, b_fc2)
    yperm = _stage_d(blk.reshape(NW), na.reshape(16), hperm, wperm,
                     We1, be1, We2, be2)
    out = _stage_e(out1, yperm, s0, s1)
    return out.reshape(B, S, D)


# speedup vs baseline: 1.3602x; 1.0069x over previous
import jax
import jax.numpy as jnp
from jax import lax
from jax.experimental import pallas as pl
from jax.experimental.pallas import tpu as pltpu
from jax.experimental.pallas import tpu_sc as plsc

B, S, D = 1, 2048, 768
DFF = 3072
E = 8
DH = 768
TEMP = 1.0
EPS = 1e-5

T_BLK = 512
NB = S // T_BLK
G_BLK = 512
NSLOT = 2 * S + E * G_BLK
ND = NSLOT // G_BLK
NW = 32
TPW = S // NW
BF = jnp.bfloat16
F32 = jnp.float32
I32 = jnp.int32


def _route_body(x_ref, lng_ref, lnb_ref, wr_ref, br_ref,
                h_ref, w1_ref, w2_ref, e0_ref, e1_ref, r0_ref, r1_ref,
                cnt_ref, run_ref):
    @pl.when(pl.program_id(0) == 0)
    def _():
        run_ref[...] = jnp.zeros((1, 16), F32)

    xb = x_ref[...]
    mu = jnp.mean(xb, axis=-1, keepdims=True)
    var = jnp.mean((xb - mu) ** 2, axis=-1, keepdims=True)
    hb = (xb - mu) / jnp.sqrt(var + EPS) * lng_ref[...] + lnb_ref[...]
    h_ref[...] = hb

    logits = jnp.dot(hb.astype(BF), wr_ref[...].astype(BF),
                     preferred_element_type=F32) + br_ref[...]
    idx = lax.broadcasted_iota(I32, (T_BLK, E), 1)
    m1 = jnp.max(logits, axis=-1, keepdims=True)
    i1 = jnp.min(jnp.where(logits == m1, idx, E), axis=-1, keepdims=True)
    mask1 = idx == i1
    rest = jnp.where(mask1, -jnp.inf, logits)
    m2 = jnp.max(rest, axis=-1, keepdims=True)
    i2 = jnp.min(jnp.where(rest == m2, idx, E), axis=-1, keepdims=True)
    mask2 = idx == i2
    w1 = 1.0 / (1.0 + jnp.exp((m2 - m1) / TEMP))

    m1f = mask1.astype(F32)
    m2f = mask2.astype(F32)
    mm = m1f + m2f
    ri = lax.broadcasted_iota(I32, (T_BLK, T_BLK), 0)
    ci = lax.broadcasted_iota(I32, (T_BLK, T_BLK), 1)
    ltri = (ci < ri).astype(BF)
    csum = jnp.dot(ltri, mm.astype(BF), preferred_element_type=F32)
    run = run_ref[...][:, :E]
    cg = csum + run
    r0 = jnp.sum(jnp.where(mask1, cg, 0.0), axis=-1)
    r1 = jnp.sum(jnp.where(mask2, cg + m1f, 0.0), axis=-1)
    newrun = run + jnp.sum(mm, axis=0, keepdims=True)
    run_ref[:, :E] = newrun
    cnt_ref[:, :E] = newrun.astype(I32)
    cnt_ref[:, E:] = jnp.zeros((1, 16 - E), I32)

    ones128 = jnp.ones((1, 128), F32)
    w1_ref[...] = w1 * ones128
    w2_ref[...] = (1.0 - w1) * ones128
    e0_ref[...] = i1.astype(I32).reshape(1, 1, T_BLK)
    e1_ref[...] = i2.astype(I32).reshape(1, 1, T_BLK)
    r0_ref[...] = r0.astype(I32).reshape(1, 1, T_BLK)
    r1_ref[...] = r1.astype(I32).reshape(1, 1, T_BLK)


def _stage_a(x, ln_g, ln_b, wr, br):
    full = lambda shape: pl.BlockSpec(shape, lambda t: (0,) * len(shape))
    per_tok = lambda dt: jax.ShapeDtypeStruct((NB, 1, T_BLK), dt)
    return pl.pallas_call(
        _route_body,
        grid=(NB,),
        in_specs=[
            pl.BlockSpec((T_BLK, D), lambda t: (t, 0)),
            full((1, D)), full((1, D)), full((D, E)), full((1, E)),
        ],
        out_specs=[
            pl.BlockSpec((T_BLK, D), lambda t: (t, 0)),
            pl.BlockSpec((T_BLK, 128), lambda t: (t, 0)),
            pl.BlockSpec((T_BLK, 128), lambda t: (t, 0)),
            pl.BlockSpec((1, 1, T_BLK), lambda t: (t, 0, 0)),
            pl.BlockSpec((1, 1, T_BLK), lambda t: (t, 0, 0)),
            pl.BlockSpec((1, 1, T_BLK), lambda t: (t, 0, 0)),
            pl.BlockSpec((1, 1, T_BLK), lambda t: (t, 0, 0)),
            full((1, 16)),
        ],
        out_shape=[
            jax.ShapeDtypeStruct((S, D), F32),
            jax.ShapeDtypeStruct((S, 128), F32),
            jax.ShapeDtypeStruct((S, 128), F32),
            per_tok(I32), per_tok(I32), per_tok(I32), per_tok(I32),
            jax.ShapeDtypeStruct((1, 16), I32),
        ],
        scratch_shapes=[pltpu.VMEM((1, 16), F32)],
        compiler_params=pltpu.CompilerParams(
            dimension_semantics=("arbitrary",)),
    )(x, ln_g.reshape(1, D), ln_b.reshape(1, D), wr, br.reshape(1, E))


def _shared_body(x_ref, h_ref, w1_ref, b1_ref, w2_ref, b2_ref, o_ref):
    hbb = h_ref[...].astype(BF)
    t1 = jnp.dot(hbb, w1_ref[...].astype(BF),
                 preferred_element_type=F32) + b1_ref[...]
    a1 = jax.nn.gelu(t1.astype(BF))
    sh = jnp.dot(a1, w2_ref[...].astype(BF),
                 preferred_element_type=F32) + b2_ref[...]
    o_ref[...] = x_ref[...] + sh


def _stage_b(x, h, wfc1, bfc1, wfc2, bfc2):
    full = lambda shape: pl.BlockSpec(shape, lambda t: (0,) * len(shape))
    return pl.pallas_call(
        _shared_body,
        grid=(NB,),
        in_specs=[
            pl.BlockSpec((T_BLK, D), lambda t: (t, 0)),
            pl.BlockSpec((T_BLK, D), lambda t: (t, 0)),
            full((D, DFF)), full((1, DFF)), full((DFF, D)), full((1, D)),
        ],
        out_specs=pl.BlockSpec((T_BLK, D), lambda t: (t, 0)),
        out_shape=jax.ShapeDtypeStruct((S, D), F32),
        compiler_params=pltpu.CompilerParams(
            dimension_semantics=("arbitrary",)),
    )(x, h, wfc1, bfc1.reshape(1, DFF), wfc2, bfc2.reshape(1, D))


def _slots_body(cnt_ref, e0_ref, e1_ref, r0_ref, r1_ref,
                s0_ref, s1_ref, blk_ref, na_ref):
    cntb = cnt_ref[...][:, :E].astype(F32)
    pb = jnp.floor((cntb + (G_BLK - 1)) * (1.0 / G_BLK))
    ri = lax.broadcasted_iota(I32, (E, E), 0)
    ci = lax.broadcasted_iota(I32, (E, E), 1)
    utri = (ri < ci).astype(BF)
    baseb = jnp.dot(pb.astype(BF), utri, preferred_element_type=F32)

    e0 = e0_ref[...].reshape(1, T_BLK)
    e1 = e1_ref[...].reshape(1, T_BLK)
    s0 = r0_ref[...].reshape(1, T_BLK).astype(F32)
    s1 = r1_ref[...].reshape(1, T_BLK).astype(F32)
    iota8 = lax.broadcasted_iota(I32, (1, E), 1)
    bstart = lax.broadcasted_iota(I32, (1, NW), 1).astype(F32)
    blk = jnp.zeros((1, NW), F32)
    na = jnp.zeros((1, 16), F32)
    for e in range(E):
        be = jnp.sum(jnp.where(iota8 == e, baseb, 0.0), axis=-1,
                     keepdims=True)
        pe = jnp.sum(jnp.where(iota8 == e, pb, 0.0), axis=-1, keepdims=True)
        s0 = s0 + jnp.where(e0 == e, be * G_BLK, 0.0)
        s1 = s1 + jnp.where(e1 == e, be * G_BLK, 0.0)
        blk = blk + jnp.where(bstart >= be, 1.0, 0.0)
        na = na + pe
    s0_ref[...] = s0.astype(I32).reshape(1, 1, T_BLK)
    s1_ref[...] = s1.astype(I32).reshape(1, 1, T_BLK)
    blk_ref[...] = (blk - 1.0).astype(I32)
    na_ref[...] = na.astype(I32)


def _stage_a2(cnts, e0o, e1o, r0o, r1o):
    full = lambda shape: pl.BlockSpec(shape, lambda t: (0,) * len(shape))
    tok = pl.BlockSpec((1, 1, T_BLK), lambda t: (t, 0, 0))
    return pl.pallas_call(
        _slots_body,
        grid=(NB,),
        in_specs=[full((1, 16)), tok, tok, tok, tok],
        out_specs=[tok, tok, full((1, NW)), full((1, 16))],
        out_shape=[
            jax.ShapeDtypeStruct((NB, 1, T_BLK), I32),
            jax.ShapeDtypeStruct((NB, 1, T_BLK), I32),
            jax.ShapeDtypeStruct((1, NW), I32),
            jax.ShapeDtypeStruct((1, 16), I32),
        ],
        compiler_params=pltpu.CompilerParams(
            dimension_semantics=("arbitrary",)),
    )(cnts, e0o, e1o, r0o, r1o)


def _dispatch_body(h_hbm, wr0_hbm, wr1_hbm, s0_hbm, s1_hbm,
                   hperm_hbm, wperm_hbm,
                   s0v, s1v, hrows, w0rows, w1rows,
                   sem0, sem1, sem2, sem3):
    wid = lax.axis_index("s") * 2 + lax.axis_index("c")
    base = wid * TPW
    pltpu.sync_copy(s0_hbm.at[pl.ds(base, TPW)], s0v)
    pltpu.sync_copy(s1_hbm.at[pl.ds(base, TPW)], s1v)
    pltpu.sync_copy(h_hbm.at[pl.ds(base, TPW)], hrows)
    pltpu.sync_copy(wr0_hbm.at[pl.ds(base, TPW)], w0rows)
    pltpu.sync_copy(wr1_hbm.at[pl.ds(base, TPW)], w1rows)
    d0 = pltpu.async_copy(hrows, hperm_hbm.at[s0v], sem0)
    d1 = pltpu.async_copy(hrows, hperm_hbm.at[s1v], sem1)
    d2 = pltpu.async_copy(w0rows, wperm_hbm.at[s0v], sem2)
    d3 = pltpu.async_copy(w1rows, wperm_hbm.at[s1v], sem3)
    d0.wait()
    d1.wait()
    d2.wait()
    d3.wait()


def _stage_c(h, wr0, wr1, s0, s1):
    mesh = plsc.VectorSubcoreMesh(core_axis_name="c", subcore_axis_name="s", num_cores=2, num_subcores=16)
    f = pl.kernel(
        _dispatch_body,
        out_type=[
            jax.ShapeDtypeStruct((NSLOT, D), F32),
            jax.ShapeDtypeStruct((NSLOT, 128), F32),
        ],
        mesh=mesh,
        compiler_params=pltpu.CompilerParams(needs_layout_passes=False),
        scratch_types=[
            pltpu.VMEM((TPW,), I32), pltpu.VMEM((TPW,), I32),
            pltpu.VMEM((TPW, D), F32),
            pltpu.VMEM((TPW, 128), F32), pltpu.VMEM((TPW, 128), F32),
            pltpu.SemaphoreType.DMA, pltpu.SemaphoreType.DMA,
            pltpu.SemaphoreType.DMA, pltpu.SemaphoreType.DMA,
        ],
    )
    return f(h, wr0, wr1, s0, s1)


def _group_body(blk_sref, na_sref, hp_ref, wp_ref, w1_ref, b1_ref, w2_ref,
                b2_ref, y_ref):
    b = pl.program_id(0)

    @pl.when(b < na_sref[0])
    def _():
        xb = hp_ref[...].astype(BF)
        t1 = jnp.dot(xb, w1_ref[0].astype(BF),
                     preferred_element_type=F32) + b1_ref[0, 0]
        a1 = jax.nn.gelu(t1.astype(BF))
        y = jnp.dot(a1, w2_ref[0].astype(BF),
                    preferred_element_type=F32) + b2_ref[0, 0]
        y_ref[...] = y * wp_ref[...][:, 0:1]


def _stage_d(blk, na, hperm, wperm, we1, be1, we2, be2):
    def beff(b, blk_r, na_r):
        return jnp.minimum(b, na_r[0] - 1)

    grid_spec = pltpu.PrefetchScalarGridSpec(
        num_scalar_prefetch=2,
        grid=(ND,),
        in_specs=[
            pl.BlockSpec((G_BLK, D), lambda b, bl, na: (beff(b, bl, na), 0)),
            pl.BlockSpec((G_BLK, 128),
                         lambda b, bl, na: (beff(b, bl, na), 0)),
            pl.BlockSpec((1, D, DH),
                         lambda b, bl, na: (bl[beff(b, bl, na)], 0, 0)),
            pl.BlockSpec((1, 1, DH),
                         lambda b, bl, na: (bl[beff(b, bl, na)], 0, 0)),
            pl.BlockSpec((1, DH, D),
                         lambda b, bl, na: (bl[beff(b, bl, na)], 0, 0)),
            pl.BlockSpec((1, 1, D),
                         lambda b, bl, na: (bl[beff(b, bl, na)], 0, 0)),
        ],
        out_specs=pl.BlockSpec((G_BLK, D),
                               lambda b, bl, na: (beff(b, bl, na), 0)),
    )
    return pl.pallas_call(
        _group_body,
        grid_spec=grid_spec,
        out_shape=jax.ShapeDtypeStruct((NSLOT, D), F32),
        compiler_params=pltpu.CompilerParams(
            dimension_semantics=("arbitrary",)),
    )(blk, na, hperm, wperm, we1, be1.reshape(E, 1, DH), we2,
      be2.reshape(E, 1, D))


_EQ = 16
_ENQ = TPW // _EQ


def _combine_body(o1_hbm, yp_hbm, s0_hbm, s1_hbm, out_hbm,
                  s0v, s1v, ob0, ob1, y00, y01, y10, y11,
                  so0, so1, sg00, sg01, sg10, sg11):
    wid = lax.axis_index("s") * 2 + lax.axis_index("c")
    base = wid * TPW
    obufs, y0s, y1s = [ob0, ob1], [y00, y01], [y10, y11]
    semo, sem0s, sem1s = [so0, so1], [sg00, sg01], [sg10, sg11]

    pltpu.sync_copy(s0_hbm.at[pl.ds(base, TPW)], s0v)
    pltpu.sync_copy(s1_hbm.at[pl.ds(base, TPW)], s1v)

    def issue(q):
        i = q % 2
        qb = base + q * _EQ
        do = pltpu.async_copy(o1_hbm.at[pl.ds(qb, _EQ)], obufs[i], semo[i])
        d0 = pltpu.async_copy(yp_hbm.at[s0v.at[pl.ds(q * _EQ, _EQ)]],
                              y0s[i], sem0s[i])
        d1 = pltpu.async_copy(yp_hbm.at[s1v.at[pl.ds(q * _EQ, _EQ)]],
                              y1s[i], sem1s[i])
        return (do, d0, d1)

    pending = issue(0)
    for q in range(_ENQ):
        nxt = issue(q + 1) if q + 1 < _ENQ else None
        for dsc in pending:
            dsc.wait()
        i = q % 2
        buf, yy0, yy1 = obufs[i], y0s[i], y1s[i]

        def row_body(r, _):
            for c in range(D // 16):
                sl = pl.ds(c * 16, 16)
                buf[r, sl] = buf[r, sl] + yy0[r, sl] + yy1[r, sl]
            return 0

        lax.fori_loop(0, _EQ, row_body, 0)
        pltpu.sync_copy(buf, out_hbm.at[pl.ds(base + q * _EQ, _EQ)])
        pending = nxt


def _stage_e(out1, yperm, s0, s1):
    mesh = plsc.VectorSubcoreMesh(core_axis_name="c", subcore_axis_name="s", num_cores=2, num_subcores=16)
    f = pl.kernel(
        _combine_body,
        out_type=jax.ShapeDtypeStruct((S, D), F32),
        mesh=mesh,
        compiler_params=pltpu.CompilerParams(needs_layout_passes=False),
        scratch_types=[
            pltpu.VMEM((TPW,), I32), pltpu.VMEM((TPW,), I32),
            pltpu.VMEM((_EQ, D), F32), pltpu.VMEM((_EQ, D), F32),
            pltpu.VMEM((_EQ, D), F32), pltpu.VMEM((_EQ, D), F32),
            pltpu.VMEM((_EQ, D), F32), pltpu.VMEM((_EQ, D), F32),
            pltpu.SemaphoreType.DMA, pltpu.SemaphoreType.DMA,
            pltpu.SemaphoreType.DMA, pltpu.SemaphoreType.DMA,
            pltpu.SemaphoreType.DMA, pltpu.SemaphoreType.DMA,
        ],
    )
    return f(out1, yperm, s0, s1)


@jax.jit
def kernel(hidden_states, ln_g, ln_b, W_fc1, b_fc1, W_fc2, b_fc2,
           W_router, b_router, We1, be1, We2, be2):
    x = hidden_states.reshape(S, D)
    (h, wr0, wr1, e0o, e1o, r0o, r1o, cnts) = _stage_a(
        x, ln_g, ln_b, W_router, b_router)

    s0o, s1o, blk, na = _stage_a2(cnts, e0o, e1o, r0o, r1o)
    s0 = s0o.reshape(S)
    s1 = s1o.reshape(S)
    hperm, wperm = _stage_c(h, wr0, wr1, s0, s1)
    out1 = _stage_b(x, h, W_fc1, b_fc1, W_fc2, b_fc2)
    yperm = _stage_d(blk.reshape(NW), na.reshape(16), hperm, wperm,
                     We1, be1, We2, be2)
    out = _stage_e(out1, yperm, s0, s1)
    return out.reshape(B, S, D)
